# Initial kernel scaffold; baseline (speedup 1.0000x reference)
import functools

import jax
import jax.numpy as jnp
from jax import lax
from jax.experimental import pallas as pl
from jax.experimental.pallas import tpu as pltpu
from jax.experimental.pallas import tpu_sc as plsc

N = 10000
E = 320000
D = 128

NC = 2
NS = 16
NW = NC * NS
EPT = E // NW
CH = 80
NCH = EPT // CH
RPT = N // NS

_mesh = plsc.VectorSubcoreMesh(core_axis_name="c", subcore_axis_name="s")


@functools.partial(
    pl.kernel,
    out_type=jax.ShapeDtypeStruct((NC, N, 16), jnp.float32),
    mesh=_mesh,
    scratch_types=[
        pltpu.VMEM((CH,), jnp.int32),
        pltpu.VMEM((CH, 16), jnp.float32),
        pltpu.VMEM((RPT, 16), jnp.float32),
        pltpu.VMEM_SHARED((N, 16), jnp.float32),
    ],
)
def _deg_kernel(dstm, degp, didx_v, ones_v, zrow_v, deg_sh):
    c = lax.axis_index("c")
    s = lax.axis_index("s")
    wid = s * NC + c

    def fill(i, _):
        ones_v[i, :] = jnp.ones((16,), jnp.float32)
        return _

    lax.fori_loop(0, CH, fill, None)

    def zfill(i, _):
        zrow_v[i, :] = jnp.zeros((16,), jnp.float32)
        return _

    lax.fori_loop(0, RPT, zfill, None)
    pltpu.sync_copy(zrow_v, deg_sh.at[pl.ds(s * RPT, RPT)])
    plsc.subcore_barrier()

    def chunk(ci, _):
        pltpu.sync_copy(dstm.at[wid, pl.ds(ci * CH, CH)], didx_v)
        pltpu.sync_copy(ones_v, deg_sh.at[didx_v], add=True)
        return _

    lax.fori_loop(0, NCH, chunk, None)
    plsc.subcore_barrier()
    pltpu.sync_copy(deg_sh.at[pl.ds(s * RPT, RPT)],
                    degp.at[c, pl.ds(s * RPT, RPT)])


_BLK = 1000


def _scale_body(feat_ref, degp_ref, out_ref):
    deg = degp_ref[0] + degp_ref[1]
    d = jnp.maximum(deg[:, 0:1], 1.0)
    out_ref[...] = feat_ref[...] * lax.rsqrt(d)


_scale_call = pl.pallas_call(
    _scale_body,
    grid=(N // _BLK,),
    in_specs=[
        pl.BlockSpec((_BLK, D), lambda i: (i, 0)),
        pl.BlockSpec((NC, _BLK, 16), lambda i: (0, i, 0)),
    ],
    out_specs=pl.BlockSpec((_BLK, D), lambda i: (i, 0)),
    out_shape=jax.ShapeDtypeStruct((N, D), jnp.float32),
)


@functools.partial(
    pl.kernel,
    out_type=jax.ShapeDtypeStruct((NC, N, D), jnp.float32),
    mesh=_mesh,
    scratch_types=[
        pltpu.VMEM((CH,), jnp.int32),
        pltpu.VMEM((CH,), jnp.int32),
        pltpu.VMEM((CH, D), jnp.float32),
        pltpu.VMEM((RPT, D), jnp.float32),
        pltpu.VMEM_SHARED((N, D), jnp.float32),
    ],
)
def _agg_kernel(scaled, srcm, dstm, part, sidx_v, didx_v, rows_v, zrow_v,
                agg_sh):
    c = lax.axis_index("c")
    s = lax.axis_index("s")
    wid = s * NC + c

    def zfill(i, _):
        for j in range(D // 16):
            zrow_v[i, pl.ds(j * 16, 16)] = jnp.zeros((16,), jnp.float32)
        return _

    lax.fori_loop(0, RPT, zfill, None)
    pltpu.sync_copy(zrow_v, agg_sh.at[pl.ds(s * RPT, RPT)])
    plsc.subcore_barrier()

    def chunk(ci, _):
        pltpu.sync_copy(srcm.at[wid, pl.ds(ci * CH, CH)], sidx_v)
        pltpu.sync_copy(dstm.at[wid, pl.ds(ci * CH, CH)], didx_v)
        pltpu.sync_copy(scaled.at[sidx_v], rows_v)
        pltpu.sync_copy(rows_v, agg_sh.at[didx_v], add=True)
        return _

    lax.fori_loop(0, NCH, chunk, None)
    plsc.subcore_barrier()
    pltpu.sync_copy(agg_sh.at[pl.ds(s * RPT, RPT)],
                    part.at[c, pl.ds(s * RPT, RPT)])


def _mm_body(part_ref, w_ref, out_ref):
    a = part_ref[0] + part_ref[1]
    out_ref[...] = lax.dot_general(a, w_ref[...], (((1,), (1,)), ((), ())),
                                   preferred_element_type=jnp.float32)


_mm_call = pl.pallas_call(
    _mm_body,
    grid=(N // _BLK,),
    in_specs=[
        pl.BlockSpec((NC, _BLK, D), lambda i: (0, i, 0)),
        pl.BlockSpec((D, D), lambda i: (0, 0)),
    ],
    out_specs=pl.BlockSpec((_BLK, D), lambda i: (i, 0)),
    out_shape=jax.ShapeDtypeStruct((N, D), jnp.float32),
)


def kernel(features, edge_index, W):
    srcm = edge_index[0].reshape(NW, EPT)
    dstm = edge_index[1].reshape(NW, EPT)
    degp = _deg_kernel(dstm)
    scaled = _scale_call(features, degp)
    part = _agg_kernel(srcm, dstm, scaled)
    return _mm_call(part, W)


# speedup vs baseline: 4.6147x; 4.6147x over previous
import functools

import jax
import jax.numpy as jnp
from jax import lax
from jax.experimental import pallas as pl
from jax.experimental.pallas import tpu as pltpu
from jax.experimental.pallas import tpu_sc as plsc

N = 10000
E = 320000
D = 128

NC = 2
NS = 16
NW = NC * NS
CH = 80
NCH = E // (NW * CH)
NP = 10240
RPT = NP // NS

_mesh = plsc.VectorSubcoreMesh(core_axis_name="c", subcore_axis_name="s")


def _deg_body(dst1, degp, didx_v, ones_v, deg_sh):
    c = lax.axis_index("c")
    s = lax.axis_index("s")
    wid = s * NC + c

    def zfill(i, _):
        for j in range(D // 16):
            ones_v[i, pl.ds(j * 16, 16)] = jnp.zeros((16,), jnp.float32)
        return _

    lax.fori_loop(0, CH, zfill, None)

    def zslab(i, _):
        pltpu.sync_copy(ones_v, deg_sh.at[pl.ds(s * RPT + i * CH, CH)])
        return _

    lax.fori_loop(0, RPT // CH, zslab, None)

    def fill(i, _):
        for j in range(D // 16):
            ones_v[i, pl.ds(j * 16, 16)] = jnp.ones((16,), jnp.float32)
        return _

    lax.fori_loop(0, CH, fill, None)
    plsc.subcore_barrier()

    def chunk(ci, _):
        base = (ci * NW + wid) * CH
        pltpu.sync_copy(dst1.at[pl.ds(base, CH)], didx_v)
        pltpu.sync_copy(ones_v, deg_sh.at[didx_v], add=True)
        return _

    lax.fori_loop(0, NCH, chunk, None)
    plsc.subcore_barrier()
    pltpu.sync_copy(deg_sh.at[pl.ds(s * RPT, RPT)],
                    degp.at[c, pl.ds(s * RPT, RPT)])


_DEG_SCRATCH = [
    pltpu.VMEM((CH,), jnp.int32),
    pltpu.VMEM((CH, D), jnp.float32),
    pltpu.VMEM_SHARED((NP, D), jnp.float32),
]

_deg_kernel = pl.kernel(
    _deg_body,
    out_type=jax.ShapeDtypeStruct((NC, NP, D), jnp.float32),
    mesh=_mesh,
    scratch_types=_DEG_SCRATCH,
)


_BLK = 1000


def _scale_body(feat_ref, degp_ref, out_ref):
    deg = degp_ref[0] + degp_ref[1]
    d = jnp.maximum(deg[:, 0:1], 1.0)
    out_ref[...] = feat_ref[...] * lax.rsqrt(d)


_scale_call = pl.pallas_call(
    _scale_body,
    grid=(N // _BLK,),
    in_specs=[
        pl.BlockSpec((_BLK, D), lambda i: (i, 0)),
        pl.BlockSpec((NC, _BLK, D), lambda i: (0, i, 0)),
    ],
    out_specs=pl.BlockSpec((_BLK, D), lambda i: (i, 0)),
    out_shape=jax.ShapeDtypeStruct((N, D), jnp.float32),
)


def _agg_body(scaled, src1, dst1, part, sidx_v, didx_v, rows_v, agg_sh):
    c = lax.axis_index("c")
    s = lax.axis_index("s")
    wid = s * NC + c

    def zfill(i, _):
        for j in range(D // 16):
            rows_v[i, pl.ds(j * 16, 16)] = jnp.zeros((16,), jnp.float32)
        return _

    lax.fori_loop(0, CH, zfill, None)

    def zslab(i, _):
        pltpu.sync_copy(rows_v, agg_sh.at[pl.ds(s * RPT + i * CH, CH)])
        return _

    lax.fori_loop(0, RPT // CH, zslab, None)
    plsc.subcore_barrier()

    def chunk(ci, _):
        base = (ci * NW + wid) * CH
        pltpu.sync_copy(src1.at[pl.ds(base, CH)], sidx_v)
        pltpu.sync_copy(dst1.at[pl.ds(base, CH)], didx_v)
        pltpu.sync_copy(scaled.at[sidx_v], rows_v)
        pltpu.sync_copy(rows_v, agg_sh.at[didx_v], add=True)
        return _

    lax.fori_loop(0, NCH, chunk, None)
    plsc.subcore_barrier()
    pltpu.sync_copy(agg_sh.at[pl.ds(s * RPT, RPT)],
                    part.at[c, pl.ds(s * RPT, RPT)])


_AGG_SCRATCH = [
    pltpu.VMEM((CH,), jnp.int32),
    pltpu.VMEM((CH,), jnp.int32),
    pltpu.VMEM((CH, D), jnp.float32),
    pltpu.VMEM_SHARED((NP, D), jnp.float32),
]

_agg_kernel = pl.kernel(
    _agg_body,
    out_type=jax.ShapeDtypeStruct((NC, NP, D), jnp.float32),
    mesh=_mesh,
    scratch_types=_AGG_SCRATCH,
)


def _mm_body(part_ref, w_ref, out_ref):
    a = part_ref[0] + part_ref[1]
    out_ref[...] = lax.dot_general(a, w_ref[...], (((1,), (1,)), ((), ())),
                                   preferred_element_type=jnp.float32)


_mm_call = pl.pallas_call(
    _mm_body,
    grid=(N // _BLK,),
    in_specs=[
        pl.BlockSpec((NC, _BLK, D), lambda i: (0, i, 0)),
        pl.BlockSpec((D, D), lambda i: (0, 0)),
    ],
    out_specs=pl.BlockSpec((_BLK, D), lambda i: (i, 0)),
    out_shape=jax.ShapeDtypeStruct((N, D), jnp.float32),
)


def kernel(features, edge_index, W):
    src1 = edge_index[0]
    dst1 = edge_index[1]
    degp = _deg_kernel(dst1)
    scaled = _scale_call(features, degp)
    part = _agg_kernel(scaled, src1, dst1)
    return _mm_call(part, W)


# speedup vs baseline: 10.4280x; 2.2597x over previous
import jax
import jax.numpy as jnp
from jax import lax
from jax.experimental import pallas as pl
from jax.experimental.pallas import tpu as pltpu
from jax.experimental.pallas import tpu_sc as plsc

N = 10000
E = 320000
D = 128

NC = 2
NS = 16
NW = NC * NS
CH = 80
NCH = E // (NW * CH)
NP = 10240
RPT = NP // NS

_mesh = plsc.VectorSubcoreMesh(core_axis_name="c", subcore_axis_name="s")


def _deg_body(dst1, degp, didx, ones_v, si, ss, deg_sh):
    c = lax.axis_index("c")
    s = lax.axis_index("s")
    wid = s * NC + c

    def zfill(i, _):
        for j in range(D // 16):
            ones_v[i, pl.ds(j * 16, 16)] = jnp.zeros((16,), jnp.float32)
        return _

    lax.fori_loop(0, CH, zfill, None)

    def zslab(i, _):
        pltpu.sync_copy(ones_v, deg_sh.at[pl.ds(s * RPT + i * CH, CH)])
        return _

    lax.fori_loop(0, RPT // CH, zslab, None)

    def fill(i, _):
        for j in range(D // 16):
            ones_v[i, pl.ds(j * 16, 16)] = jnp.ones((16,), jnp.float32)
        return _

    lax.fori_loop(0, CH, fill, None)
    plsc.subcore_barrier()

    def idx_desc(q, ci):
        base = (ci * NW + wid) * CH
        return pltpu.make_async_copy(dst1.at[pl.ds(base, CH)], didx[q], si[q])

    def sct_desc(r):
        return pltpu.make_async_copy(ones_v, deg_sh.at[didx[r]], ss[r])

    idx_desc(0, 0).start()
    idx_desc(1, 1).start()

    def step(ci, r):
        idx_desc(r, ci).wait()

        @pl.when(ci >= 6)
        def _():
            sct_desc((r - 6) % DRING).wait()

        @pl.when(ci < NCH - 2)
        def _():
            idx_desc((r + 2) % DRING, ci + 2).start()

        sct_desc(r).start(add=True)

    def chunk(ci, _):
        for k in range(DRING):
            @pl.when(ci % DRING == k)
            def _():
                step(ci, k)
        return _

    lax.fori_loop(0, NCH, chunk, None)
    for j in range(NCH - 6, NCH):
        sct_desc(j % DRING).wait()

    plsc.subcore_barrier()
    pltpu.sync_copy(deg_sh.at[pl.ds(s * RPT, RPT)],
                    degp.at[c, pl.ds(s * RPT, RPT)])


DRING = 8

_DEG_SCRATCH = [
    [pltpu.VMEM((CH,), jnp.int32) for _ in range(DRING)],
    pltpu.VMEM((CH, D), jnp.float32),
    [pltpu.SemaphoreType.DMA for _ in range(DRING)],
    [pltpu.SemaphoreType.DMA for _ in range(DRING)],
    pltpu.VMEM_SHARED((NP, D), jnp.float32),
]

_deg_kernel = pl.kernel(
    _deg_body,
    out_type=jax.ShapeDtypeStruct((NC, NP, D), jnp.float32),
    mesh=_mesh,
    scratch_types=_DEG_SCRATCH,
)


_BLK = 1000


def _scale_body(feat_ref, degp_ref, out_ref):
    deg = degp_ref[0, :, 0:1] + degp_ref[1, :, 0:1]
    d = jnp.maximum(deg, 1.0)
    out_ref[...] = feat_ref[...] * lax.rsqrt(d)


_scale_call = pl.pallas_call(
    _scale_body,
    grid=(N // _BLK,),
    in_specs=[
        pl.BlockSpec((_BLK, D), lambda i: (i, 0)),
        pl.BlockSpec((NC, _BLK, D), lambda i: (0, i, 0)),
    ],
    out_specs=pl.BlockSpec((_BLK, D), lambda i: (i, 0)),
    out_shape=jax.ShapeDtypeStruct((N, D), jnp.float32),
)


def _agg_body(scaled, src1, dst1, part, sidx, didx, rows, si, sg, ss,
              agg_sh):
    c = lax.axis_index("c")
    s = lax.axis_index("s")
    wid = s * NC + c

    def zfill(i, _):
        for j in range(D // 16):
            rows[0][i, pl.ds(j * 16, 16)] = jnp.zeros((16,), jnp.float32)
        return _

    lax.fori_loop(0, CHA, zfill, None)

    def zslab(i, _):
        pltpu.sync_copy(rows[0], agg_sh.at[pl.ds(s * RPT + i * CHA, CHA)])
        return _

    lax.fori_loop(0, RPT // CHA, zslab, None)
    plsc.subcore_barrier()

    def idx_desc(r, ci):
        base = (ci * NW + wid) * CHA
        return (
            pltpu.make_async_copy(src1.at[pl.ds(base, CHA)], sidx[r], si[r]),
            pltpu.make_async_copy(dst1.at[pl.ds(base, CHA)], didx[r], si[r]),
        )

    def gat_desc(r):
        return pltpu.make_async_copy(scaled.at[sidx[r]], rows[r], sg[r])

    def sct_desc(r):
        return pltpu.make_async_copy(rows[r], agg_sh.at[didx[r]], ss[r])

    def idx_start(r, ci):
        d1, d2 = idx_desc(r, ci)
        d1.start()
        d2.start()

    def idx_wait(r, ci):
        d1, d2 = idx_desc(r, ci)
        d1.wait()
        d2.wait()

    idx_start(0, 0)
    idx_start(1, 1)

    def step(ci, r):
        idx_wait(r, ci)

        @pl.when(ci >= 6)
        def _():
            sct_desc((r - 6) % RING).wait()

        @pl.when(ci < NCHA - 2)
        def _():
            idx_start((r + 2) % RING, ci + 2)

        gat_desc(r).start()

        @pl.when(ci >= GL)
        def _():
            rr = (r - GL) % RING
            gat_desc(rr).wait()
            sct_desc(rr).start(add=True)

    def chunk(ci, _):
        for k in range(RING):
            @pl.when(ci % RING == k)
            def _():
                step(ci, k)
        return _

    lax.fori_loop(0, NCHA, chunk, None)
    for j in range(NCHA - GL, NCHA):
        gat_desc(j % RING).wait()
        sct_desc(j % RING).start(add=True)
    for j in range(NCHA - 6, NCHA):
        sct_desc(j % RING).wait()

    plsc.subcore_barrier()
    pltpu.sync_copy(agg_sh.at[pl.ds(s * RPT, RPT)],
                    part.at[c, pl.ds(s * RPT, RPT)])


CHA = 40
NCHA = E // (NW * CHA)
RING = 8
GL = 3

_AGG_SCRATCH = [
    [pltpu.VMEM((CHA,), jnp.int32) for _ in range(RING)],
    [pltpu.VMEM((CHA,), jnp.int32) for _ in range(RING)],
    [pltpu.VMEM((CHA, D), jnp.float32) for _ in range(RING)],
    [pltpu.SemaphoreType.DMA for _ in range(RING)],
    [pltpu.SemaphoreType.DMA for _ in range(RING)],
    [pltpu.SemaphoreType.DMA for _ in range(RING)],
    pltpu.VMEM_SHARED((NP, D), jnp.float32),
]

_agg_kernel = pl.kernel(
    _agg_body,
    out_type=jax.ShapeDtypeStruct((NC, NP, D), jnp.float32),
    mesh=_mesh,
    scratch_types=_AGG_SCRATCH,
)


def _mm_body(part_ref, w_ref, out_ref):
    a = part_ref[0] + part_ref[1]
    out_ref[...] = lax.dot_general(a, w_ref[...], (((1,), (1,)), ((), ())),
                                   preferred_element_type=jnp.float32)


_mm_call = pl.pallas_call(
    _mm_body,
    grid=(N // _BLK,),
    in_specs=[
        pl.BlockSpec((NC, _BLK, D), lambda i: (0, i, 0)),
        pl.BlockSpec((D, D), lambda i: (0, 0)),
    ],
    out_specs=pl.BlockSpec((_BLK, D), lambda i: (i, 0)),
    out_shape=jax.ShapeDtypeStruct((N, D), jnp.float32),
)


def kernel(features, edge_index, W):
    src1 = edge_index[0]
    dst1 = edge_index[1]
    degp = _deg_kernel(dst1)
    scaled = _scale_call(features, degp)
    part = _agg_kernel(scaled, src1, dst1)
    return _mm_call(part, W)
